# baseline (speedup 1.0000x reference)
import jax
import jax.numpy as jnp
from jax import lax
from jax.experimental import pallas as pl
from jax.experimental.pallas import tpu as pltpu
from jax.experimental.pallas import tpu_sc as plsc

N = 10000
E = 320000
D = 128
NC = 2
NS = 16
NW = NC * NS
CH = 96
CPW = 108
E2 = NW * CPW * CH
NACC = N + 8
NTAB = N + 16
RPT = N // NS
DPT = 640
NDEN = NS * DPT
ZR = 25


def _pre_body(hs_ref, hd_ref, wl_ref, wr_ref, al_ref, ar_ref):
    al_ref[...] = jnp.dot(hs_ref[...], wl_ref[...],
                          preferred_element_type=jnp.float32)
    ar_ref[...] = jnp.dot(hd_ref[...], wr_ref[...],
                          preferred_element_type=jnp.float32)


def _post_body(acc_ref, d0_ref, d1_ref, b_ref, o_ref):
    num = acc_ref[0] + acc_ref[1]
    den = d0_ref[...] + d1_ref[...]
    o_ref[...] = num / den + jnp.broadcast_to(b_ref[...], (N, D))


def _sc_body(hs, al_h, ar_h, src_h, dst_h, out, dout,
             acc, den, al_v, ar_v, srcv, dstv, wbuf, rows_v, zbuf,
             gsem0, gsem1, ssem0, ssem1,
             isem0, isem1, isem2, isem3):
    cid = lax.axis_index("c")
    sid = lax.axis_index("s")
    wid = sid * NC + cid
    gsems = (gsem0, gsem1)
    ssems = (ssem0, ssem1)
    isems = (isem0, isem1, isem2, isem3)

    def _idx_fire(t, iph):
        base = (t * NW + wid) * CH
        pltpu.async_copy(src_h.at[pl.ds(base, CH)], srcv.at[iph], isems[iph])
        pltpu.async_copy(dst_h.at[pl.ds(base, CH)], dstv.at[iph], isems[iph])

    def _idx_wait(t, iph):
        base = (t * NW + wid) * CH
        pltpu.make_async_copy(src_h.at[pl.ds(base, CH)], srcv.at[iph],
                              isems[iph]).wait()
        pltpu.make_async_copy(dst_h.at[pl.ds(base, CH)], dstv.at[iph],
                              isems[iph]).wait()

    def _fire(t, ph, iph, drain, pref):
        if drain:
            pltpu.make_async_copy(rows_v.at[ph], acc.at[dstv.at[(iph + 2) % 4]],
                                  ssems[ph]).wait()
            pltpu.make_async_copy(wbuf.at[ph], den.at[dstv.at[(iph + 2) % 4]],
                                  ssems[ph]).wait()
        _idx_wait(t, iph)
        pltpu.async_copy(hs.at[srcv.at[iph]], rows_v.at[ph], gsems[ph])
        if pref:
            _idx_fire(t + 2, (iph + 2) % 4)

    def _process(t, ph, iph):
        pltpu.make_async_copy(hs.at[srcv.at[iph]], rows_v.at[ph],
                              gsems[ph]).wait()

        def _scale(i, cc):
            si = srcv[iph, pl.ds(i * 16, 16)]
            di = dstv[iph, pl.ds(i * 16, 16)]
            s = plsc.load_gather(al_v, [si]) + plsc.load_gather(ar_v, [di])
            s = jnp.where(s >= 0, s, s * jnp.float32(0.01))
            wv = jnp.exp(s)
            wbuf[ph, pl.ds(i * 16, 16)] = wv
            for l in range(16):
                e = i * 16 + l
                w = jnp.broadcast_to(wv[l], (16,))
                for j in range(D // 16):
                    rows_v[ph, e, pl.ds(j * 16, 16)] = (
                        rows_v[ph, e, pl.ds(j * 16, 16)] * w)
            return cc
        lax.fori_loop(0, CH // 16, _scale, 0)

        pltpu.async_copy(rows_v.at[ph], acc.at[dstv.at[iph]], ssems[ph],
                         add=True)
        pltpu.async_copy(wbuf.at[ph], den.at[dstv.at[iph]], ssems[ph],
                         add=True)

    _idx_fire(0, 0)
    _idx_fire(1, 1)
    _fire(0, 0, 0, False, True)
    _fire(1, 1, 1, False, True)

    pltpu.sync_copy(al_h, al_v)
    pltpu.sync_copy(ar_h, ar_v)

    def _zrow(i, c):
        for j in range(D // 16):
            zbuf[i, pl.ds(j * 16, 16)] = jnp.zeros((16,), jnp.float32)
        return c
    lax.fori_loop(0, ZR, _zrow, 0)

    row0 = sid * RPT
    for k in range(RPT // ZR):
        pltpu.sync_copy(zbuf, acc.at[pl.ds(row0 + k * ZR, ZR)])
    for k in range(DPT // D):
        pltpu.sync_copy(zbuf.at[0], den.at[pl.ds(sid * DPT + k * D, D)])
    plsc.subcore_barrier()

    def _loop(t4, c):
        t0 = 4 * t4
        for ci in range(4):
            t = t0 + ci
            _process(t, ci % 2, ci)
            _fire(t + 2, ci % 2, (ci + 2) % 4, True, True)
        return c
    lax.fori_loop(0, (CPW - 4) // 4, _loop, 0)

    for t in range(CPW - 4, CPW - 2):
        _process(t, t % 2, t % 4)
        _fire(t + 2, t % 2, (t + 2) % 4, True, False)
    for t in range(CPW - 2, CPW):
        _process(t, t % 2, t % 4)
        pltpu.make_async_copy(rows_v.at[t % 2], acc.at[dstv.at[t % 4]],
                              ssems[t % 2]).wait()
        pltpu.make_async_copy(wbuf.at[t % 2], den.at[dstv.at[t % 4]],
                              ssems[t % 2]).wait()

    plsc.subcore_barrier()
    pltpu.sync_copy(acc.at[pl.ds(row0, RPT)], out.at[cid, pl.ds(row0, RPT)])
    pltpu.sync_copy(den.at[pl.ds(sid * DPT, DPT)],
                    dout.at[cid, pl.ds(sid * DPT, DPT)])


@jax.jit
def kernel(h_src, h_dst, edge_index, attn_l, attn_r, bias):
    al, ar = pl.pallas_call(
        _pre_body,
        out_shape=(jax.ShapeDtypeStruct((N, 1), jnp.float32),
                   jax.ShapeDtypeStruct((N, 1), jnp.float32)),
    )(h_src, h_dst, attn_l.reshape(D, 1), attn_r.reshape(D, 1))

    pad16 = jnp.zeros((16,), jnp.float32)
    alp = jnp.concatenate([al.reshape(N), pad16])
    arp = jnp.concatenate([ar.reshape(N), pad16])

    srcp = jnp.concatenate(
        [edge_index[0], jnp.zeros((E2 - E,), jnp.int32)])
    dstp = jnp.concatenate(
        [edge_index[1],
         N + (jnp.arange(E2 - E, dtype=jnp.int32) % 8)])

    sc = pl.kernel(
        _sc_body,
        out_type=(jax.ShapeDtypeStruct((NC, N, D), jnp.float32),
                  jax.ShapeDtypeStruct((NC, NDEN), jnp.float32)),
        mesh=plsc.VectorSubcoreMesh(core_axis_name="c", subcore_axis_name="s"),
        compiler_params=pltpu.CompilerParams(use_tc_tiling_on_sc=False,
                                             needs_layout_passes=False),
        scratch_types=[
            pltpu.VMEM_SHARED((NACC, D), jnp.float32),
            pltpu.VMEM_SHARED((NDEN,), jnp.float32),
            pltpu.VMEM((NTAB,), jnp.float32),
            pltpu.VMEM((NTAB,), jnp.float32),
            pltpu.VMEM((4, CH), jnp.int32),
            pltpu.VMEM((4, CH), jnp.int32),
            pltpu.VMEM((2, CH), jnp.float32),
            pltpu.VMEM((2, CH, D), jnp.float32),
            pltpu.VMEM((ZR, D), jnp.float32),
            pltpu.SemaphoreType.DMA,
            pltpu.SemaphoreType.DMA,
            pltpu.SemaphoreType.DMA,
            pltpu.SemaphoreType.DMA,
            pltpu.SemaphoreType.DMA,
            pltpu.SemaphoreType.DMA,
            pltpu.SemaphoreType.DMA,
            pltpu.SemaphoreType.DMA,
        ],
    )
    acc, dsum = sc(h_src, alp, arp, srcp, dstp)

    d0 = dsum[0, :N].reshape(N, 1)
    d1 = dsum[1, :N].reshape(N, 1)
    return pl.pallas_call(
        _post_body,
        out_shape=jax.ShapeDtypeStruct((N, D), jnp.float32),
    )(acc, d0, d1, bias.reshape(1, D))
